# baseline (speedup 1.0000x reference)
import functools

import jax
import jax.numpy as jnp
import numpy as np
from jax import lax
from jax.experimental import pallas as pl
from jax.experimental.pallas import tpu as pltpu
from jax.experimental.pallas import tpu_sc as plsc

NC = 2
NS = 16
NW = NC * NS
L = 16

CHUNK = 160


@functools.partial(jax.jit, static_argnames=("vocab", "embed"))
def _sc_format_table(tableT, ttail, *, vocab, embed):
    width = 2 * embed
    nblk = vocab // width
    per_w = nblk // NW
    rem = nblk - per_w * NW
    mesh = plsc.VectorSubcoreMesh(core_axis_name="c", subcore_axis_name="s")

    @functools.partial(
        pl.kernel,
        out_type=jax.ShapeDtypeStruct((vocab // 2, width), jnp.float32),
        mesh=mesh,
        compiler_params=pltpu.CompilerParams(use_tc_tiling_on_sc=True,
                                             needs_layout_passes=False),
        scratch_types=[
            pltpu.VMEM((embed, width), jnp.float32),
            pltpu.VMEM((embed, width), jnp.float32),
            pltpu.VMEM((embed, width), jnp.float32),
            pltpu.SemaphoreType.DMA,
            pltpu.SemaphoreType.DMA,
        ],
    )
    def k(tableT_hbm, ttail_hbm, out_hbm, slab0, slab1, obuf, sem0, sem1):
        c = lax.axis_index("c")
        s = lax.axis_index("s")
        wid = s * NC + c
        n_w = per_w + jnp.where(wid < rem, 1, 0)
        blk0 = wid * per_w + jnp.minimum(wid, rem)
        slabs = (slab0, slab1)
        sems = (sem0, sem1)

        def start_read(g, b):
            pltpu.async_copy(
                tableT_hbm.at[:, pl.ds((blk0 + g) * width, width)],
                slabs[b], sems[b])

        def transpose_block(g, b):
            slab = slabs[b]

            def row(kk, _):
                for half in range(2):
                    col = jnp.broadcast_to(2 * kk + half, (L,))
                    for l in range(embed // L):
                        rows = l * L + lax.iota(jnp.int32, L)
                        obuf[kk, pl.ds(half * embed + l * L, L)] = (
                            plsc.load_gather(slab, [rows, col]))
                return ()
            lax.fori_loop(0, width // 2, row, (), unroll=2)
            pltpu.sync_copy(
                obuf, out_hbm.at[pl.ds((blk0 + g) * (width // 2), width // 2)])

        @pl.when(n_w > 0)
        def _():
            start_read(0, 0)

            def pair(h, _):
                g0 = h * 2
                pltpu.make_async_copy(tableT_hbm, slab0, sem0).wait()

                @pl.when(g0 + 1 < n_w)
                def _():
                    start_read(g0 + 1, 1)
                transpose_block(g0, 0)

                @pl.when(g0 + 1 < n_w)
                def _():
                    pltpu.make_async_copy(tableT_hbm, slab1, sem1).wait()

                    @pl.when(g0 + 2 < n_w)
                    def _():
                        start_read(g0 + 2, 0)
                    transpose_block(g0 + 1, 1)
                return ()

            lax.fori_loop(0, (n_w + 1) // 2, pair, (), unroll=False)

        @pl.when(wid == 0)
        def _():
            pltpu.sync_copy(ttail_hbm.at[pl.ds(0, (vocab - nblk * width) // 2)],
                            slab0.at[pl.ds(0, (vocab - nblk * width) // 2)])
            pltpu.sync_copy(slab0.at[pl.ds(0, (vocab - nblk * width) // 2)],
                            out_hbm.at[pl.ds(nblk * (width // 2),
                                             (vocab - nblk * width) // 2)])

    return k(tableT, ttail)


@functools.partial(jax.jit, static_argnames=("batch", "seq", "embed"))
def _sc_gather_pool(x_flat, tpair, *, batch, seq, embed):
    elems_per_w = batch // NW
    rows_per_w = elems_per_w * seq
    nchunks = rows_per_w // CHUNK
    assert nchunks % 2 == 0
    acc_rows = 2 * elems_per_w
    mesh = plsc.VectorSubcoreMesh(core_axis_name="c", subcore_axis_name="s")

    @functools.partial(
        pl.kernel,
        out_type=jax.ShapeDtypeStruct((batch, embed), jnp.float32),
        mesh=mesh,
        compiler_params=pltpu.CompilerParams(use_tc_tiling_on_sc=True),
        scratch_types=[
            pltpu.VMEM((rows_per_w,), jnp.int32),
            pltpu.VMEM((CHUNK, 2 * embed), jnp.float32),
            pltpu.VMEM((CHUNK, 2 * embed), jnp.float32),
            pltpu.VMEM((CHUNK,), jnp.int32),
            pltpu.VMEM((CHUNK,), jnp.int32),
            pltpu.VMEM((CHUNK,), jnp.int32),
            pltpu.VMEM((CHUNK,), jnp.int32),
            pltpu.VMEM((elems_per_w, embed), jnp.float32),
            pltpu.VMEM_SHARED((NS * acc_rows, 2 * embed), jnp.float32),
            pltpu.SemaphoreType.DMA,
            pltpu.SemaphoreType.DMA,
        ],
    )
    def k(x_hbm, tpair_hbm, out_hbm,
          idx_v, buf0, buf1, gidx0, gidx1, dst0, dst1, out_v, acc_sh,
          sem0, sem1):
        c = lax.axis_index("c")
        s = lax.axis_index("s")
        wid = s * NC + c
        row_base = wid * rows_per_w
        bufs = (buf0, buf1)
        sems = (sem0, sem1)
        gidxs = (gidx0, gidx1)
        dsts = (dst0, dst1)

        pltpu.sync_copy(x_hbm.at[pl.ds(row_base, rows_per_w)], idx_v)

        zero = jnp.zeros((L,), jnp.float32)

        def zrow(r, _):
            for l in range(2 * embed // L):
                buf0[r, pl.ds(l * L, L)] = zero
            return ()
        lax.fori_loop(0, CHUNK, zrow, (), unroll=False)
        pltpu.sync_copy(buf0, acc_sh.at[pl.ds(s * acc_rows, CHUNK)])
        pltpu.sync_copy(buf0.at[pl.ds(0, acc_rows - CHUNK)],
                        acc_sh.at[pl.ds(s * acc_rows + CHUNK, acc_rows - CHUNK)])

        def prep(i, b):
            def body(t, _):
                pos = i * CHUNK + t * L
                raw = idx_v[pl.ds(pos, L)]
                posv = pos + lax.iota(jnp.int32, L)
                base = s * acc_rows + 2 * lax.div(posv, seq)
                gidxs[b][pl.ds(t * L, L)] = lax.shift_right_logical(raw, 1)
                dsts[b][pl.ds(t * L, L)] = base + lax.bitwise_and(raw, 1)
                return ()
            lax.fori_loop(0, CHUNK // L, body, (), unroll=True)

        def start_gather(b):
            pltpu.async_copy(tpair_hbm.at[gidxs[b]], bufs[b], sems[b])

        def pool(b):
            pltpu.sync_copy(bufs[b], acc_sh.at[dsts[b]], add=True)

        prep(0, 0)
        start_gather(0)

        def pair(g, _):
            i0 = g * 2
            pltpu.make_async_copy(tpair_hbm, buf0, sem0).wait()
            prep(i0 + 1, 1)
            start_gather(1)
            pool(0)
            pltpu.make_async_copy(tpair_hbm, buf1, sem1).wait()
            prep(i0 + 2, 0)
            start_gather(0)
            pool(1)
            return ()

        lax.fori_loop(0, nchunks // 2 - 1, pair, (), unroll=False)

        pltpu.make_async_copy(tpair_hbm, buf0, sem0).wait()
        prep(nchunks - 1, 1)
        start_gather(1)
        pool(0)
        pltpu.make_async_copy(tpair_hbm, buf1, sem1).wait()
        pool(1)

        pltpu.sync_copy(acc_sh.at[pl.ds(s * acc_rows, CHUNK)], buf0)
        pltpu.sync_copy(acc_sh.at[pl.ds(s * acc_rows + CHUNK, acc_rows - CHUNK)],
                        buf1.at[pl.ds(0, acc_rows - CHUNK)])

        def mkfix(buf, e0):
            def fix(e, _):
                for l in range(embed // L):
                    out_v[e0 + e, pl.ds(l * L, L)] = (
                        buf[2 * e, pl.ds(l * L, L)]
                        + buf[2 * e + 1, pl.ds(embed + l * L, L)])
                return ()
            return fix
        lax.fori_loop(0, CHUNK // 2, mkfix(buf0, 0), (), unroll=False)
        lax.fori_loop(0, (acc_rows - CHUNK) // 2, mkfix(buf1, CHUNK // 2), (),
                      unroll=False)

        pltpu.sync_copy(out_v, out_hbm.at[pl.ds(wid * elems_per_w, elems_per_w)])

    return k(x_flat, tpair)


def _dense_softmax(pooled_sum, W, b2, inv_seq, block_b):
    batch, embed = pooled_sum.shape
    out = W.shape[1]

    def body(p_ref, w_ref, b_ref, o_ref):
        logits = jnp.dot(p_ref[...] * inv_seq, w_ref[...],
                         preferred_element_type=jnp.float32) + b_ref[...]
        m = jnp.max(logits, axis=-1, keepdims=True)
        e = jnp.exp(logits - m)
        o_ref[...] = e / jnp.sum(e, axis=-1, keepdims=True)

    return pl.pallas_call(
        body,
        grid=(batch // block_b,),
        in_specs=[
            pl.BlockSpec((block_b, embed), lambda i: (i, 0)),
            pl.BlockSpec((embed, out), lambda i: (0, 0)),
            pl.BlockSpec((1, out), lambda i: (0, 0)),
        ],
        out_specs=pl.BlockSpec((block_b, out), lambda i: (i, 0)),
        out_shape=jax.ShapeDtypeStruct((batch, out), jnp.float32),
    )(pooled_sum, W, b2)


def kernel(x, table, W, b):
    batch, seq = x.shape
    vocab, embed = table.shape
    elems_per_w = batch // NW
    rows_per_w = elems_per_w * seq

    nblk = vocab // (2 * embed)
    ttail = table[nblk * 2 * embed:].reshape(-1, 2 * embed)
    tpair = _sc_format_table(table.T, ttail, vocab=vocab, embed=embed)

    pooled_sum = _sc_gather_pool(x.reshape(-1), tpair,
                                 batch=batch, seq=seq, embed=embed)
    return _dense_softmax(pooled_sum, W, b.reshape(1, -1), 1.0 / seq, 256)


# speedup vs baseline: 2.2277x; 2.2277x over previous
import functools

import jax
import jax.numpy as jnp
import numpy as np
from jax import lax
from jax.experimental import pallas as pl
from jax.experimental.pallas import tpu as pltpu
from jax.experimental.pallas import tpu_sc as plsc

NC = 2
NS = 16
NW = NC * NS
L = 16

CHUNK = 160


@functools.partial(jax.jit, static_argnames=("vocab", "embed"))
def _sc_format_table(tableT, ttail, *, vocab, embed):
    width = 2 * embed
    nblk = vocab // width
    per_w = nblk // NW
    rem = nblk - per_w * NW
    mesh = plsc.VectorSubcoreMesh(core_axis_name="c", subcore_axis_name="s")

    @functools.partial(
        pl.kernel,
        out_type=jax.ShapeDtypeStruct((vocab // 2, width), jnp.float32),
        mesh=mesh,
        compiler_params=pltpu.CompilerParams(use_tc_tiling_on_sc=True,
                                             needs_layout_passes=False),
        scratch_types=[
            pltpu.VMEM((embed, width), jnp.float32),
            pltpu.VMEM((embed, width), jnp.float32),
            pltpu.VMEM((embed, width), jnp.float32),
            pltpu.SemaphoreType.DMA,
            pltpu.SemaphoreType.DMA,
        ],
    )
    def k(tableT_hbm, ttail_hbm, out_hbm, slab0, slab1, obuf, sem0, sem1):
        c = lax.axis_index("c")
        s = lax.axis_index("s")
        wid = s * NC + c
        n_w = per_w + jnp.where(wid < rem, 1, 0)
        blk0 = wid * per_w + jnp.minimum(wid, rem)
        slabs = (slab0, slab1)
        sems = (sem0, sem1)

        def start_read(g, b):
            pltpu.async_copy(
                tableT_hbm.at[:, pl.ds((blk0 + g) * width, width)],
                slabs[b], sems[b])

        def transpose_block(g, b):
            slab = slabs[b]

            def row(kk, _):
                for half in range(2):
                    col = jnp.broadcast_to(2 * kk + half, (L,))
                    for l in range(embed // L):
                        rows = l * L + lax.iota(jnp.int32, L)
                        obuf[kk, pl.ds(half * embed + l * L, L)] = (
                            plsc.load_gather(slab, [rows, col]))
                return ()
            lax.fori_loop(0, width // 2, row, (), unroll=2)
            pltpu.sync_copy(
                obuf, out_hbm.at[pl.ds((blk0 + g) * (width // 2), width // 2)])

        @pl.when(n_w > 0)
        def _():
            start_read(0, 0)

            def pair(h, _):
                g0 = h * 2
                pltpu.make_async_copy(tableT_hbm, slab0, sem0).wait()

                @pl.when(g0 + 1 < n_w)
                def _():
                    start_read(g0 + 1, 1)
                transpose_block(g0, 0)

                @pl.when(g0 + 1 < n_w)
                def _():
                    pltpu.make_async_copy(tableT_hbm, slab1, sem1).wait()

                    @pl.when(g0 + 2 < n_w)
                    def _():
                        start_read(g0 + 2, 0)
                    transpose_block(g0 + 1, 1)
                return ()

            lax.fori_loop(0, (n_w + 1) // 2, pair, (), unroll=False)

        @pl.when(wid == 0)
        def _():
            pltpu.sync_copy(ttail_hbm.at[pl.ds(0, (vocab - nblk * width) // 2)],
                            slab0.at[pl.ds(0, (vocab - nblk * width) // 2)])
            pltpu.sync_copy(slab0.at[pl.ds(0, (vocab - nblk * width) // 2)],
                            out_hbm.at[pl.ds(nblk * (width // 2),
                                             (vocab - nblk * width) // 2)])

    return k(tableT, ttail)


@functools.partial(jax.jit, static_argnames=("batch", "seq", "embed"))
def _sc_gather_pool(x_flat, tpair, *, batch, seq, embed):
    elems_per_w = batch // NW
    rows_per_w = elems_per_w * seq
    nchunks = rows_per_w // CHUNK
    assert nchunks % 2 == 0
    acc_rows = elems_per_w
    mesh = plsc.VectorSubcoreMesh(core_axis_name="c", subcore_axis_name="s")

    @functools.partial(
        pl.kernel,
        out_type=jax.ShapeDtypeStruct((batch, embed), jnp.float32),
        mesh=mesh,
        compiler_params=pltpu.CompilerParams(use_tc_tiling_on_sc=True),
        scratch_types=[
            pltpu.VMEM((rows_per_w,), jnp.int32),
            pltpu.VMEM((CHUNK, 2 * embed), jnp.float32),
            pltpu.VMEM((CHUNK, 2 * embed), jnp.float32),
            pltpu.VMEM((CHUNK,), jnp.int32),
            pltpu.VMEM((CHUNK,), jnp.int32),
            pltpu.VMEM((CHUNK,), jnp.int32),
            pltpu.VMEM((CHUNK,), jnp.int32),
            pltpu.VMEM((elems_per_w, embed), jnp.float32),
            pltpu.VMEM_SHARED((NS * acc_rows, 2 * embed), jnp.float32),
            pltpu.SemaphoreType.DMA,
            pltpu.SemaphoreType.DMA,
        ],
    )
    def k(x_hbm, tpair_hbm, out_hbm,
          idx_v, buf0, buf1, gidx0, gidx1, dst0, dst1, out_v, acc_sh,
          sem0, sem1):
        c = lax.axis_index("c")
        s = lax.axis_index("s")
        wid = s * NC + c
        row_base = wid * rows_per_w
        bufs = (buf0, buf1)
        sems = (sem0, sem1)
        gidxs = (gidx0, gidx1)
        dsts = (dst0, dst1)

        pltpu.sync_copy(x_hbm.at[pl.ds(row_base, rows_per_w)], idx_v)

        zero = jnp.zeros((L,), jnp.float32)

        def zrow(r, _):
            for l in range(2 * embed // L):
                buf0[r, pl.ds(l * L, L)] = zero
            return ()
        lax.fori_loop(0, CHUNK, zrow, (), unroll=False)
        pltpu.sync_copy(buf0.at[pl.ds(0, acc_rows)],
                        acc_sh.at[pl.ds(s * acc_rows, acc_rows)])

        def prep(i, b):
            def body(t, _):
                pos = i * CHUNK + t * L
                raw = idx_v[pl.ds(pos, L)]
                posv = pos + lax.iota(jnp.int32, L)
                gidxs[b][pl.ds(t * L, L)] = raw
                dsts[b][pl.ds(t * L, L)] = s * acc_rows + lax.div(posv, seq)
                return ()
            lax.fori_loop(0, CHUNK // L, body, (), unroll=True)

        def start_gather(b):
            pltpu.async_copy(tpair_hbm.at[gidxs[b]], bufs[b], sems[b])

        def pool(b):
            pltpu.sync_copy(bufs[b], acc_sh.at[dsts[b]], add=True)

        prep(0, 0)
        start_gather(0)

        def pair(g, _):
            i0 = g * 2
            pltpu.make_async_copy(tpair_hbm, buf0, sem0).wait()
            prep(i0 + 1, 1)
            start_gather(1)
            pool(0)
            pltpu.make_async_copy(tpair_hbm, buf1, sem1).wait()
            prep(i0 + 2, 0)
            start_gather(0)
            pool(1)
            return ()

        lax.fori_loop(0, nchunks // 2 - 1, pair, (), unroll=False)

        pltpu.make_async_copy(tpair_hbm, buf0, sem0).wait()
        prep(nchunks - 1, 1)
        start_gather(1)
        pool(0)
        pltpu.make_async_copy(tpair_hbm, buf1, sem1).wait()
        pool(1)

        pltpu.sync_copy(acc_sh.at[pl.ds(s * acc_rows, acc_rows)],
                        buf0.at[pl.ds(0, acc_rows)])

        def fix(e, _):
            for l in range(embed // L):
                out_v[e, pl.ds(l * L, L)] = buf0[e, pl.ds(l * L, L)]
            return ()
        lax.fori_loop(0, acc_rows, fix, (), unroll=False)

        pltpu.sync_copy(out_v, out_hbm.at[pl.ds(wid * elems_per_w, elems_per_w)])

    return k(x_flat, tpair)


def _dense_softmax(pooled_sum, W, b2, inv_seq, block_b):
    batch, embed = pooled_sum.shape
    out = W.shape[1]

    def body(p_ref, w_ref, b_ref, o_ref):
        logits = jnp.dot(p_ref[...] * inv_seq, w_ref[...],
                         preferred_element_type=jnp.float32) + b_ref[...]
        m = jnp.max(logits, axis=-1, keepdims=True)
        e = jnp.exp(logits - m)
        o_ref[...] = e / jnp.sum(e, axis=-1, keepdims=True)

    return pl.pallas_call(
        body,
        grid=(batch // block_b,),
        in_specs=[
            pl.BlockSpec((block_b, embed), lambda i: (i, 0)),
            pl.BlockSpec((embed, out), lambda i: (0, 0)),
            pl.BlockSpec((1, out), lambda i: (0, 0)),
        ],
        out_specs=pl.BlockSpec((block_b, out), lambda i: (i, 0)),
        out_shape=jax.ShapeDtypeStruct((batch, out), jnp.float32),
    )(pooled_sum, W, b2)


def kernel(x, table, W, b):
    batch, seq = x.shape
    vocab, embed = table.shape
    elems_per_w = batch // NW
    rows_per_w = elems_per_w * seq

    tpad = jnp.pad(table, ((0, 0), (0, 2 * embed - table.shape[1])))

    pooled_sum = _sc_gather_pool(x.reshape(-1), tpad,
                                 batch=batch, seq=seq, embed=embed)
    return _dense_softmax(pooled_sum, W, b.reshape(1, -1), 1.0 / seq, 256)


# speedup vs baseline: 2.3721x; 1.0648x over previous
import functools

import jax
import jax.numpy as jnp
import numpy as np
from jax import lax
from jax.experimental import pallas as pl
from jax.experimental.pallas import tpu as pltpu
from jax.experimental.pallas import tpu_sc as plsc

NC = 2
NS = 16
NW = NC * NS

CHUNK_ROWS = 512


@functools.partial(jax.jit, static_argnames=("batch", "seq", "embed"))
def _sc_gather_pool(x_flat, table, dst_pat, zeros, *, batch, seq, embed):
    elems_per_w = batch // NW
    rows_per_w = elems_per_w * seq
    nchunks = rows_per_w // CHUNK_ROWS
    assert nchunks % 2 == 0
    mesh = plsc.VectorSubcoreMesh(core_axis_name="c", subcore_axis_name="s")

    @functools.partial(
        pl.kernel,
        out_type=jax.ShapeDtypeStruct((batch, embed), jnp.float32),
        mesh=mesh,
        compiler_params=pltpu.CompilerParams(use_tc_tiling_on_sc=False),
        scratch_types=[
            pltpu.VMEM((rows_per_w,), jnp.int32),
            pltpu.VMEM((CHUNK_ROWS, embed), jnp.float32),
            pltpu.VMEM((CHUNK_ROWS, embed), jnp.float32),
            pltpu.VMEM((nchunks, CHUNK_ROWS), jnp.int32),
            pltpu.VMEM_SHARED((NS * elems_per_w, embed), jnp.float32),
            pltpu.SemaphoreType.DMA,
            pltpu.SemaphoreType.DMA,
        ],
    )
    def k(x_hbm, table_hbm, dstpat_hbm, zeros_hbm, out_hbm,
          idx_v, buf0, buf1, dst_v, acc_sh, sem0, sem1):
        c = lax.axis_index("c")
        s = lax.axis_index("s")
        wid = s * NC + c
        row_base = wid * rows_per_w
        bufs = (buf0, buf1)
        sems = (sem0, sem1)

        pltpu.sync_copy(x_hbm.at[pl.ds(row_base, rows_per_w)], idx_v)
        pltpu.sync_copy(dstpat_hbm.at[wid], dst_v)
        pltpu.sync_copy(zeros_hbm, acc_sh.at[pl.ds(s * elems_per_w, elems_per_w)])

        def start_gather(i, b):
            pltpu.async_copy(
                table_hbm.at[idx_v.at[pl.ds(i * CHUNK_ROWS, CHUNK_ROWS)]],
                bufs[b], sems[b])

        def pool(i, b):
            pltpu.sync_copy(bufs[b], acc_sh.at[dst_v.at[i]], add=True)

        start_gather(0, 0)

        def pair(g, _):
            i0 = g * 2
            pltpu.make_async_copy(table_hbm, buf0, sem0).wait()
            start_gather(i0 + 1, 1)
            pool(i0, 0)
            pltpu.make_async_copy(table_hbm, buf1, sem1).wait()
            start_gather(i0 + 2, 0)
            pool(i0 + 1, 1)
            return ()

        lax.fori_loop(0, nchunks // 2 - 1, pair, (), unroll=False)

        pltpu.make_async_copy(table_hbm, buf0, sem0).wait()
        start_gather(nchunks - 1, 1)
        pool(nchunks - 2, 0)
        pltpu.make_async_copy(table_hbm, buf1, sem1).wait()
        pool(nchunks - 1, 1)

        pltpu.sync_copy(acc_sh.at[pl.ds(s * elems_per_w, elems_per_w)],
                        out_hbm.at[pl.ds(wid * elems_per_w, elems_per_w)])

    return k(x_flat, table, dst_pat, zeros)


def _dense_softmax(pooled_sum, W, b2, inv_seq, block_b):
    batch, embed = pooled_sum.shape
    out = W.shape[1]

    def body(p_ref, w_ref, b_ref, o_ref):
        logits = jnp.dot(p_ref[...] * inv_seq, w_ref[...],
                         preferred_element_type=jnp.float32) + b_ref[...]
        m = jnp.max(logits, axis=-1, keepdims=True)
        e = jnp.exp(logits - m)
        o_ref[...] = e / jnp.sum(e, axis=-1, keepdims=True)

    return pl.pallas_call(
        body,
        grid=(batch // block_b,),
        in_specs=[
            pl.BlockSpec((block_b, embed), lambda i: (i, 0)),
            pl.BlockSpec((embed, out), lambda i: (0, 0)),
            pl.BlockSpec((1, out), lambda i: (0, 0)),
        ],
        out_specs=pl.BlockSpec((block_b, out), lambda i: (i, 0)),
        out_shape=jax.ShapeDtypeStruct((batch, out), jnp.float32),
    )(pooled_sum, W, b2)


def kernel(x, table, W, b):
    batch, seq = x.shape
    vocab, embed = table.shape
    elems_per_w = batch // NW

    e_idx = np.repeat(np.arange(elems_per_w, dtype=np.int32), seq)
    dst_pat = (e_idx[None, :] +
               (np.arange(NW, dtype=np.int32)[:, None] // NC) * elems_per_w)
    dst_pat = dst_pat.reshape(NW, -1, CHUNK_ROWS).astype(np.int32)
    zeros = jnp.zeros((elems_per_w, embed), jnp.float32)

    pooled_sum = _sc_gather_pool(x.reshape(-1), table, jnp.asarray(dst_pat),
                                 zeros, batch=batch, seq=seq, embed=embed)
    return _dense_softmax(pooled_sum, W, b.reshape(1, -1), 1.0 / seq, 256)
